# baseline (speedup 1.0000x reference)
import functools

import jax
import jax.numpy as jnp
from jax import lax
from jax.experimental import pallas as pl
from jax.experimental.pallas import tpu as pltpu
from jax.experimental.pallas import tpu_sc as plsc

_N_NODES = 10000
_N_EDGES = 320000
_D_FEAT = 128
_D_EDGE = 16

_NW = 32
_PER_W = _N_EDGES // _NW
_CE = 2000
_CHUNKS = _PER_W // _CE
_BE = 8000


def _tables_body(xn_ref, w0_ref, w1_ref, a_ref, b_ref):
    x = xn_ref[...]
    a_ref[...] = jnp.dot(x, w0_ref[...], preferred_element_type=jnp.float32)
    b_ref[...] = jnp.dot(x, w1_ref[...], preferred_element_type=jnp.float32)


def _c_body(xe_ref, w2_ref, b_ref, o_ref):
    o_ref[...] = (
        jnp.dot(xe_ref[...], w2_ref[...], preferred_element_type=jnp.float32)
        + b_ref[...]
    )


_SC_PARAMS = pltpu.CompilerParams(
    use_tc_tiling_on_sc=False, needs_layout_passes=False)


def _make_sc_gather_sum():
    mesh = plsc.VectorSubcoreMesh(core_axis_name="c", subcore_axis_name="s")

    @functools.partial(
        pl.kernel,
        mesh=mesh,
        compiler_params=_SC_PARAMS,
        out_type=jax.ShapeDtypeStruct((_N_EDGES, _D_EDGE), jnp.float32),
        scratch_types=[
            pltpu.VMEM((2 * _CE,), jnp.int32),
            pltpu.VMEM((_CE,), jnp.int32),
            pltpu.VMEM((_CE,), jnp.int32),
            pltpu.VMEM((_CE, _D_EDGE), jnp.float32),
            pltpu.SemaphoreType.DMA,
            pltpu.SemaphoreType.DMA,
        ],
    )
    def sc_gather_sum(a_hbm, b_hbm, ef_hbm, out_hbm,
                      ev, idx0, idx1, acc, sem_a, sem_b):
        wid = lax.axis_index("s") * 2 + lax.axis_index("c")
        base = wid * _PER_W
        lane = lax.broadcasted_iota(jnp.int32, (16,), 0)

        def chunk(j, carry):
            off = base + j * _CE
            pltpu.sync_copy(ef_hbm.at[pl.ds(2 * off, 2 * _CE)], ev)

            def deint(k, c2):
                ids = lane + 16 * k
                idx0[pl.ds(16 * k, 16)] = plsc.load_gather(ev, [2 * ids])
                idx1[pl.ds(16 * k, 16)] = plsc.load_gather(ev, [2 * ids + 1])
                return c2

            lax.fori_loop(0, _CE // 16, deint, 0)
            pltpu.async_copy(a_hbm.at[idx0], acc, sem_a).wait()
            pltpu.async_copy(b_hbm.at[idx1], acc, sem_b, add=True).wait()
            pltpu.sync_copy(acc, out_hbm.at[pl.ds(off, _CE)])
            return carry

        lax.fori_loop(0, _CHUNKS, chunk, 0)

    return sc_gather_sum


def _make_sc_merge():
    mesh = plsc.VectorSubcoreMesh(core_axis_name="c", subcore_axis_name="s")

    @functools.partial(
        pl.kernel,
        mesh=mesh,
        compiler_params=_SC_PARAMS,
        out_type=jax.ShapeDtypeStruct((_N_EDGES, _D_EDGE), jnp.float32),
        scratch_types=[
            pltpu.VMEM((_CE,), jnp.int32),
            pltpu.VMEM((_CE, _D_EDGE), jnp.float32),
            pltpu.SemaphoreType.DMA,
        ],
    )
    def sc_merge(g_hbm, c_hbm, out_hbm, idx, acc, sem):
        wid = lax.axis_index("s") * 2 + lax.axis_index("c")
        base = wid * _PER_W
        lane = lax.broadcasted_iota(jnp.int32, (16,), 0)

        def chunk(j, carry):
            off = base + j * _CE

            def iota_fill(k, c2):
                idx[pl.ds(16 * k, 16)] = off + lane + 16 * k
                return c2

            lax.fori_loop(0, _CE // 16, iota_fill, 0)
            pltpu.sync_copy(c_hbm.at[pl.ds(off, _CE)], acc)
            pltpu.async_copy(g_hbm.at[idx], acc, sem, add=True).wait()
            pltpu.sync_copy(acc, out_hbm.at[pl.ds(off, _CE)])
            return carry

        lax.fori_loop(0, _CHUNKS, chunk, 0)

    return sc_merge


_sc_gather_sum = _make_sc_gather_sum()
_sc_merge = _make_sc_merge()


def kernel(x_node, x_edge, edge_index, W, b):
    ef = edge_index.astype(jnp.int32).reshape(2 * _N_EDGES)
    w0 = W[:_D_FEAT]
    w1 = W[_D_FEAT:2 * _D_FEAT]
    w2 = W[2 * _D_FEAT:]

    tab_a, tab_b = pl.pallas_call(
        _tables_body,
        out_shape=[
            jax.ShapeDtypeStruct((_N_NODES, _D_EDGE), jnp.float32),
            jax.ShapeDtypeStruct((_N_NODES, _D_EDGE), jnp.float32),
        ],
    )(x_node, w0, w1)

    gsum = _sc_gather_sum(tab_a, tab_b, ef)

    c = pl.pallas_call(
        _c_body,
        grid=(_N_EDGES // _BE,),
        in_specs=[
            pl.BlockSpec((_BE, _D_EDGE), lambda i: (i, 0)),
            pl.BlockSpec((_D_EDGE, _D_EDGE), lambda i: (0, 0)),
            pl.BlockSpec((1, _D_EDGE), lambda i: (0, 0)),
        ],
        out_specs=pl.BlockSpec((_BE, _D_EDGE), lambda i: (i, 0)),
        out_shape=jax.ShapeDtypeStruct((_N_EDGES, _D_EDGE), jnp.float32),
    )(x_edge, w2, b[None, :])

    return _sc_merge(gsum, c)


# speedup vs baseline: 1.2773x; 1.2773x over previous
import functools

import jax
import jax.numpy as jnp
from jax import lax
from jax.experimental import pallas as pl
from jax.experimental.pallas import tpu as pltpu
from jax.experimental.pallas import tpu_sc as plsc

_N_NODES = 10000
_N_EDGES = 320000
_D_FEAT = 128
_D_EDGE = 16

_NW = 32
_PER_W = _N_EDGES // _NW
_CE = 2000
_CHUNKS = _PER_W // _CE


def _tables_body(xn_ref, w0_ref, w1_ref, a_ref, b_ref):
    x = xn_ref[...]
    a_ref[...] = jnp.dot(x, w0_ref[...], preferred_element_type=jnp.float32)
    b_ref[...] = jnp.dot(x, w1_ref[...], preferred_element_type=jnp.float32)


def _merge_body(xe2_ref, w2b_ref, bb_ref, g_ref, o_ref):
    o_ref[...] = (
        jnp.dot(xe2_ref[...], w2b_ref[...], preferred_element_type=jnp.float32)
        + bb_ref[...]
        + g_ref[...]
    )


def _make_sc_gather_sum():
    mesh = plsc.VectorSubcoreMesh(core_axis_name="c", subcore_axis_name="s")

    @functools.partial(
        pl.kernel,
        mesh=mesh,
        compiler_params=pltpu.CompilerParams(
            use_tc_tiling_on_sc=False, needs_layout_passes=False),
        out_type=[
            jax.ShapeDtypeStruct((_N_EDGES, _D_EDGE), jnp.float32),
            jax.ShapeDtypeStruct((_N_EDGES, _D_EDGE), jnp.float32),
        ],
        scratch_types=[
            pltpu.VMEM((2 * _CE,), jnp.int32),
            pltpu.VMEM((_CE,), jnp.int32),
            pltpu.VMEM((_CE,), jnp.int32),
            pltpu.VMEM((_CE, _D_EDGE), jnp.float32),
            pltpu.VMEM((_CE, _D_EDGE), jnp.float32),
            pltpu.SemaphoreType.DMA,
            pltpu.SemaphoreType.DMA,
            pltpu.SemaphoreType.DMA,
        ],
    )
    def sc_gather_sum(a_hbm, b_hbm, ef_hbm, xe_hbm, out_hbm, xeo_hbm,
                      ev, idx0, idx1, acc, xbuf, sem_a, sem_b, sem_x):
        wid = lax.axis_index("s") * 2 + lax.axis_index("c")
        base = wid * _PER_W
        lane = lax.broadcasted_iota(jnp.int32, (16,), 0)

        def chunk(j, carry):
            off = base + j * _CE
            cp_x = pltpu.async_copy(xe_hbm.at[pl.ds(off, _CE)], xbuf, sem_x)
            pltpu.sync_copy(ef_hbm.at[pl.ds(2 * off, 2 * _CE)], ev)

            def deint(k, c2):
                ids = lane + 16 * k
                idx0[pl.ds(16 * k, 16)] = plsc.load_gather(ev, [2 * ids])
                idx1[pl.ds(16 * k, 16)] = plsc.load_gather(ev, [2 * ids + 1])
                return c2

            lax.fori_loop(0, _CE // 16, deint, 0)
            pltpu.async_copy(a_hbm.at[idx0], acc, sem_a).wait()
            pltpu.async_copy(b_hbm.at[idx1], acc, sem_b, add=True).wait()
            pltpu.sync_copy(acc, out_hbm.at[pl.ds(off, _CE)])
            cp_x.wait()
            pltpu.sync_copy(xbuf, xeo_hbm.at[pl.ds(off, _CE)])
            return carry

        lax.fori_loop(0, _CHUNKS, chunk, 0)

    return sc_gather_sum


_sc_gather_sum = _make_sc_gather_sum()


def kernel(x_node, x_edge, edge_index, W, b):
    ef = edge_index.astype(jnp.int32).reshape(2 * _N_EDGES)
    w0 = W[:_D_FEAT]
    w1 = W[_D_FEAT:2 * _D_FEAT]
    w2 = W[2 * _D_FEAT:]

    tab_a, tab_b = pl.pallas_call(
        _tables_body,
        out_shape=[
            jax.ShapeDtypeStruct((_N_NODES, _D_EDGE), jnp.float32),
            jax.ShapeDtypeStruct((_N_NODES, _D_EDGE), jnp.float32),
        ],
    )(x_node, w0, w1)

    gsum, xe_lin = _sc_gather_sum(tab_a, tab_b, ef, x_edge)

    rows = _N_EDGES // 8
    gsum2 = gsum.reshape(rows, 128)
    xe2 = xe_lin.reshape(rows, 128)

    w2_blk = jnp.kron(jnp.eye(8, dtype=jnp.float32), w2)
    b_blk = jnp.tile(b, 8)[None, :]
    blk = rows // 8
    out2 = pl.pallas_call(
        _merge_body,
        grid=(8,),
        in_specs=[
            pl.BlockSpec((blk, 128), lambda i: (i, 0)),
            pl.BlockSpec((128, 128), lambda i: (0, 0)),
            pl.BlockSpec((1, 128), lambda i: (0, 0)),
            pl.BlockSpec((blk, 128), lambda i: (i, 0)),
        ],
        out_specs=pl.BlockSpec((blk, 128), lambda i: (i, 0)),
        out_shape=jax.ShapeDtypeStruct((rows, 128), jnp.float32),
    )(xe2, w2_blk, b_blk, gsum2)
    return out2.reshape(_N_EDGES, _D_EDGE)


# speedup vs baseline: 1.8767x; 1.4693x over previous
import functools

import jax
import jax.numpy as jnp
from jax import lax
from jax.experimental import pallas as pl
from jax.experimental.pallas import tpu as pltpu
from jax.experimental.pallas import tpu_sc as plsc

_N_NODES = 10000
_N_EDGES = 320000
_D_FEAT = 128
_D_EDGE = 16

_NW = 32
_PER_W = _N_EDGES // _NW
_CE = 2000
_CHUNKS = _PER_W // _CE


def _tables_body(xn_ref, w0_ref, w1_ref, a_ref, b_ref):
    x = xn_ref[...]
    a_ref[...] = jnp.dot(x, w0_ref[...], preferred_element_type=jnp.float32)
    b_ref[...] = jnp.dot(x, w1_ref[...], preferred_element_type=jnp.float32)


def _edge_mm_body(xe_ref, w_ref, b_ref, o_ref):
    o_ref[...] = (
        jnp.dot(xe_ref[...], w_ref[...], preferred_element_type=jnp.float32)
        + b_ref[...]
    )


def _make_sc_combine():
    mesh = plsc.VectorSubcoreMesh(core_axis_name="c", subcore_axis_name="s")

    @functools.partial(
        pl.kernel,
        mesh=mesh,
        compiler_params=pltpu.CompilerParams(use_tc_tiling_on_sc=False),
        out_type=jax.ShapeDtypeStruct((_N_EDGES, _D_EDGE), jnp.float32),
        scratch_types=[
            pltpu.VMEM((_CE,), jnp.int32),
            pltpu.VMEM((_CE,), jnp.int32),
            pltpu.VMEM((_CE, _D_EDGE), jnp.float32),
            pltpu.SemaphoreType.DMA,
            pltpu.SemaphoreType.DMA,
        ],
    )
    def sc_combine(a_hbm, b_hbm, e0_hbm, e1_hbm, c_hbm, out_hbm,
                   idx0, idx1, acc, sem_a, sem_b):
        wid = lax.axis_index("s") * 2 + lax.axis_index("c")
        base = wid * _PER_W

        def chunk(j, carry):
            off = base + j * _CE
            cp0 = pltpu.async_copy(e0_hbm.at[pl.ds(off, _CE)], idx0, sem_a)
            cp1 = pltpu.async_copy(e1_hbm.at[pl.ds(off, _CE)], idx1, sem_b)
            pltpu.sync_copy(c_hbm.at[pl.ds(off, _CE)], acc)
            cp0.wait()
            cp1.wait()
            pltpu.async_copy(a_hbm.at[idx0], acc, sem_a, add=True).wait()
            pltpu.async_copy(b_hbm.at[idx1], acc, sem_b, add=True).wait()
            pltpu.sync_copy(acc, out_hbm.at[pl.ds(off, _CE)])
            return carry

        lax.fori_loop(0, _CHUNKS, chunk, 0)

    return sc_combine


_sc_combine = _make_sc_combine()


def kernel(x_node, x_edge, edge_index, W, b):
    e = edge_index.astype(jnp.int32)
    e0 = e[:, 0]
    e1 = e[:, 1]
    w0 = W[:_D_FEAT]
    w1 = W[_D_FEAT:2 * _D_FEAT]
    w2 = W[2 * _D_FEAT:]

    tab_a, tab_b = pl.pallas_call(
        _tables_body,
        out_shape=[
            jax.ShapeDtypeStruct((_N_NODES, _D_EDGE), jnp.float32),
            jax.ShapeDtypeStruct((_N_NODES, _D_EDGE), jnp.float32),
        ],
    )(x_node, w0, w1)

    w2_blk = jnp.kron(jnp.eye(8, dtype=jnp.float32), w2)
    b_blk = jnp.tile(b, 8)[None, :]
    xe2 = x_edge.reshape(_N_EDGES // 8, 8 * _D_EDGE)
    rows = _N_EDGES // 8
    blk = rows // 8
    c2 = pl.pallas_call(
        _edge_mm_body,
        grid=(8,),
        in_specs=[
            pl.BlockSpec((blk, 8 * _D_EDGE), lambda i: (i, 0)),
            pl.BlockSpec((8 * _D_EDGE, 8 * _D_EDGE), lambda i: (0, 0)),
            pl.BlockSpec((1, 8 * _D_EDGE), lambda i: (0, 0)),
        ],
        out_specs=pl.BlockSpec((blk, 8 * _D_EDGE), lambda i: (i, 0)),
        out_shape=jax.ShapeDtypeStruct((rows, 8 * _D_EDGE), jnp.float32),
    )(xe2, w2_blk, b_blk)
    c = c2.reshape(_N_EDGES, _D_EDGE)

    return _sc_combine(tab_a, tab_b, e0, e1, c)


# speedup vs baseline: 2.0307x; 1.0821x over previous
import functools

import jax
import jax.numpy as jnp
from jax import lax
from jax.experimental import pallas as pl
from jax.experimental.pallas import tpu as pltpu
from jax.experimental.pallas import tpu_sc as plsc

_N_NODES = 10000
_N_EDGES = 320000
_D_FEAT = 128
_D_EDGE = 16

_NW = 32
_PER_W = _N_EDGES // _NW
_CE = 2000
_CHUNKS = _PER_W // _CE


def _tables_body(xn_ref, w0_ref, w1_ref, a_ref, b_ref):
    x = xn_ref[...]
    a_ref[...] = jnp.dot(x, w0_ref[...], preferred_element_type=jnp.float32)
    b_ref[...] = jnp.dot(x, w1_ref[...], preferred_element_type=jnp.float32)


def _merge_body(xe2_ref, w2b_ref, bb_ref, g_ref, o_ref):
    o_ref[...] = (
        jnp.dot(xe2_ref[...], w2b_ref[...], preferred_element_type=jnp.float32)
        + bb_ref[...]
        + g_ref[...]
    )


def _make_sc_gather_sum():
    mesh = plsc.VectorSubcoreMesh(core_axis_name="c", subcore_axis_name="s")

    @functools.partial(
        pl.kernel,
        mesh=mesh,
        compiler_params=pltpu.CompilerParams(use_tc_tiling_on_sc=False),
        out_type=jax.ShapeDtypeStruct((_N_EDGES, _D_EDGE), jnp.float32),
        scratch_types=[
            pltpu.VMEM((_CE,), jnp.int32),
            pltpu.VMEM((_CE,), jnp.int32),
            pltpu.VMEM((_CE, _D_EDGE), jnp.float32),
            pltpu.SemaphoreType.DMA,
            pltpu.SemaphoreType.DMA,
        ],
    )
    def sc_gather_sum(a_hbm, b_hbm, e0_hbm, e1_hbm, out_hbm,
                      idx0, idx1, acc, sem_a, sem_b):
        wid = lax.axis_index("s") * 2 + lax.axis_index("c")
        base = wid * _PER_W

        def chunk(j, carry):
            off = base + j * _CE
            cp0 = pltpu.async_copy(e0_hbm.at[pl.ds(off, _CE)], idx0, sem_a)
            cp1 = pltpu.async_copy(e1_hbm.at[pl.ds(off, _CE)], idx1, sem_b)
            cp0.wait()
            cp1.wait()
            pltpu.async_copy(a_hbm.at[idx0], acc, sem_a).wait()
            pltpu.async_copy(b_hbm.at[idx1], acc, sem_b, add=True).wait()
            pltpu.sync_copy(acc, out_hbm.at[pl.ds(off, _CE)])
            return carry

        lax.fori_loop(0, _CHUNKS, chunk, 0)

    return sc_gather_sum


_sc_gather_sum = _make_sc_gather_sum()


def kernel(x_node, x_edge, edge_index, W, b):
    e = edge_index.astype(jnp.int32)
    e0 = e[:, 0]
    e1 = e[:, 1]
    w0 = W[:_D_FEAT]
    w1 = W[_D_FEAT:2 * _D_FEAT]
    w2 = W[2 * _D_FEAT:]

    tab_a, tab_b = pl.pallas_call(
        _tables_body,
        out_shape=[
            jax.ShapeDtypeStruct((_N_NODES, _D_EDGE), jnp.float32),
            jax.ShapeDtypeStruct((_N_NODES, _D_EDGE), jnp.float32),
        ],
    )(x_node, w0, w1)

    gsum = _sc_gather_sum(tab_a, tab_b, e0, e1)

    rows = _N_EDGES // 8
    gsum2 = gsum.reshape(rows, 128)
    w2_blk = jnp.kron(jnp.eye(8, dtype=jnp.float32), w2)
    b_blk = jnp.tile(b, 8)[None, :]
    xe2 = x_edge.reshape(rows, 128)
    blk = rows // 8
    out2 = pl.pallas_call(
        _merge_body,
        grid=(8,),
        in_specs=[
            pl.BlockSpec((blk, 128), lambda i: (i, 0)),
            pl.BlockSpec((128, 128), lambda i: (0, 0)),
            pl.BlockSpec((1, 128), lambda i: (0, 0)),
            pl.BlockSpec((blk, 128), lambda i: (i, 0)),
        ],
        out_specs=pl.BlockSpec((blk, 128), lambda i: (i, 0)),
        out_shape=jax.ShapeDtypeStruct((rows, 128), jnp.float32),
    )(xe2, w2_blk, b_blk, gsum2)
    return out2.reshape(_N_EDGES, _D_EDGE)
